# baseline (speedup 1.0000x reference)
import functools

import jax
import jax.numpy as jnp
from jax import lax
from jax.experimental import pallas as pl
from jax.experimental.pallas import tpu as pltpu
from jax.experimental.pallas import tpu_sc as plsc

BATCH = 16
POOL = 1000
CTX_LEN = 12
CTX_DIM = 512
CLS_NUM = 100
SEQ = 77
PROMPT_LEN = 2 * CTX_LEN
SUF_LEN = SEQ - 1 - PROMPT_LEN
NC_SUF_LEN = SEQ - 1 - CTX_LEN
NC_SEQ = 1 + PROMPT_LEN + NC_SUF_LEN

_CBLK = 20
_PBLK = 40


def _sc_ctx_body(idx_hbm, gflat, aflat, ctx_out, idx_v, gidx_v, rows_v, sem):
    wid = lax.axis_index("s") * 2 + lax.axis_index("c")

    @pl.when(wid < 24)
    def _():
        pltpu.sync_copy(idx_hbm, idx_v)
        m = wid * 16 + lax.iota(jnp.int32, 16)
        k = lax.div(m, CTX_LEN)
        sel = lax.rem(k, BATCH)
        dnums = lax.GatherDimensionNumbers(
            offset_dims=(), collapsed_slice_dims=(0,), start_index_map=(0,))
        pool_row = lax.gather(
            idx_v[...], sel.reshape(16, 1), dnums, (1,),
            mode=lax.GatherScatterMode.PROMISE_IN_BOUNDS)
        gidx_v[...] = pool_row * CTX_LEN + (m - k * CTX_LEN)

        @pl.when(wid < 12)
        def _():
            pltpu.async_copy(gflat.at[gidx_v], rows_v, sem).wait()

        @pl.when(wid >= 12)
        def _():
            pltpu.async_copy(aflat.at[gidx_v], rows_v, sem).wait()

        pltpu.sync_copy(rows_v, ctx_out.at[pl.ds(wid * 16, 16)])


def _sc_gather_ctx(indices_g, gflat, aflat):
    k = functools.partial(
        pl.kernel,
        mesh=plsc.VectorSubcoreMesh(core_axis_name="c", subcore_axis_name="s"),
        out_type=jax.ShapeDtypeStruct((2 * BATCH * CTX_LEN, CTX_DIM), jnp.float32),
        scratch_types=[
            pltpu.VMEM((BATCH,), jnp.int32),
            pltpu.VMEM((16,), jnp.int32),
            pltpu.VMEM((16, CTX_DIM), jnp.float32),
            pltpu.SemaphoreType.DMA,
        ],
    )(_sc_ctx_body)
    return k(indices_g, gflat, aflat)


def _tc_prompts_body(ctx_ref, pre_ref, suf_ref, tokp_ref, out_ref, tok_ref):
    out_ref[0, :, 0:1, :] = pre_ref[...]
    out_ref[0, :, 1:1 + PROMPT_LEN, :] = jnp.broadcast_to(
        ctx_ref[...], (_CBLK, PROMPT_LEN, CTX_DIM))
    out_ref[0, :, 1 + PROMPT_LEN:SEQ, :] = suf_ref[...]
    tok_ref[0] = tokp_ref[...]


def _tc_prompts(ctx, token_prefix, token_suffix, tokenized_prompts):
    prompts, tok = pl.pallas_call(
        _tc_prompts_body,
        grid=(BATCH, CLS_NUM // _CBLK),
        in_specs=[
            pl.BlockSpec((1, PROMPT_LEN, CTX_DIM), lambda b, cb: (b, 0, 0)),
            pl.BlockSpec((_CBLK, 1, CTX_DIM), lambda b, cb: (cb, 0, 0)),
            pl.BlockSpec((_CBLK, SUF_LEN, CTX_DIM), lambda b, cb: (cb, 0, 0)),
            pl.BlockSpec((CLS_NUM, SEQ), lambda b, cb: (0, 0)),
        ],
        out_specs=[
            pl.BlockSpec((1, _CBLK, SEQ, CTX_DIM), lambda b, cb: (b, cb, 0, 0)),
            pl.BlockSpec((1, CLS_NUM, SEQ), lambda b, cb: (b, 0, 0)),
        ],
        out_shape=[
            jax.ShapeDtypeStruct((BATCH, CLS_NUM, SEQ, CTX_DIM), jnp.float32),
            jax.ShapeDtypeStruct((BATCH, CLS_NUM, SEQ), tokenized_prompts.dtype),
        ],
    )(ctx, token_prefix, token_suffix, tokenized_prompts)
    return (prompts.reshape(BATCH * CLS_NUM, SEQ, CTX_DIM),
            tok.reshape(BATCH * CLS_NUM, SEQ))


def _tc_nc_body(g_ref, a_ref, pre_ref, suf_ref, tok_ref, out_ref, nctok_ref):
    out_ref[:, 0:1, :] = jnp.broadcast_to(pre_ref[...], (_PBLK, 1, CTX_DIM))
    out_ref[:, 1:1 + CTX_LEN, :] = g_ref[...]
    out_ref[:, 1 + CTX_LEN:1 + PROMPT_LEN, :] = a_ref[...]
    out_ref[:, 1 + PROMPT_LEN:NC_SEQ, :] = jnp.broadcast_to(
        suf_ref[...], (_PBLK, NC_SUF_LEN, CTX_DIM))
    nctok_ref[...] = jnp.broadcast_to(tok_ref[...], (_PBLK, SEQ))


def _tc_nc(global_prompt, attribute_prompt, nc_token_prefix, nc_token_suffix,
           nc_tokenized_prompts):
    return pl.pallas_call(
        _tc_nc_body,
        grid=(POOL // _PBLK,),
        in_specs=[
            pl.BlockSpec((_PBLK, CTX_LEN, CTX_DIM), lambda p: (p, 0, 0)),
            pl.BlockSpec((_PBLK, CTX_LEN, CTX_DIM), lambda p: (p, 0, 0)),
            pl.BlockSpec((1, 1, CTX_DIM), lambda p: (0, 0, 0)),
            pl.BlockSpec((1, NC_SUF_LEN, CTX_DIM), lambda p: (0, 0, 0)),
            pl.BlockSpec((1, SEQ), lambda p: (0, 0)),
        ],
        out_specs=[
            pl.BlockSpec((_PBLK, NC_SEQ, CTX_DIM), lambda p: (p, 0, 0)),
            pl.BlockSpec((_PBLK, SEQ), lambda p: (p, 0)),
        ],
        out_shape=[
            jax.ShapeDtypeStruct((POOL, NC_SEQ, CTX_DIM), jnp.float32),
            jax.ShapeDtypeStruct((POOL, SEQ), nc_tokenized_prompts.dtype),
        ],
    )(global_prompt, attribute_prompt, nc_token_prefix, nc_token_suffix,
      nc_tokenized_prompts)


def kernel(indices_g, global_prompt, attribute_prompt, token_prefix,
           token_suffix, nc_token_prefix, nc_token_suffix, tokenized_prompts,
           nc_tokenized_prompts):
    idx = indices_g.astype(jnp.int32)
    gflat = global_prompt.reshape(POOL * CTX_LEN, CTX_DIM)
    aflat = attribute_prompt.reshape(POOL * CTX_LEN, CTX_DIM)

    ctx = _sc_gather_ctx(idx, gflat, aflat)
    ctx = ctx.reshape(BATCH, PROMPT_LEN, CTX_DIM)

    prompts, tok = _tc_prompts(ctx, token_prefix, token_suffix,
                               tokenized_prompts)
    nc_prompts, nc_tok = _tc_nc(global_prompt, attribute_prompt,
                                nc_token_prefix, nc_token_suffix,
                                nc_tokenized_prompts)
    return prompts, tok, nc_prompts, nc_tok
